# Initial kernel scaffold; baseline (speedup 1.0000x reference)
import functools

import jax
import jax.numpy as jnp
from jax import lax
from jax.experimental import pallas as pl
from jax.experimental.pallas import tpu as pltpu
from jax.experimental.pallas import tpu_sc as plsc

N = 10000
E = 320000
DIN = 128
DH = 128
DC = 64

NP = 10240
BLK = 512
NTILES = 32
CHUNK = 128
NCH = 80
EPT = NCH * CHUNK
EP = NTILES * EPT
RPT = NP // 16


def _sc_scatter(d):
  mesh = plsc.VectorSubcoreMesh(core_axis_name="c", subcore_axis_name="s")

  @functools.partial(
      pl.kernel,
      out_type=jax.ShapeDtypeStruct((2, NP, d), jnp.float32),
      mesh=mesh,
      scratch_types=[
          pltpu.VMEM((NCH, CHUNK), jnp.int32),
          pltpu.VMEM((NCH, CHUNK), jnp.int32),
          pltpu.VMEM((CHUNK, d), jnp.float32),
          pltpu.SemaphoreType.DMA,
          pltpu.VMEM_SHARED((NP, d), jnp.float32),
      ],
  )
  def sc_scatter(src_hbm, dst_hbm, table_hbm, zeros_hbm, out_hbm,
                 srcv, dstv, gbuf, gsem, acc):
    c = lax.axis_index("c")
    s = lax.axis_index("s")
    w = c * 16 + s
    pltpu.sync_copy(src_hbm.at[w], srcv)
    pltpu.sync_copy(dst_hbm.at[w], dstv)
    pltpu.sync_copy(zeros_hbm, acc.at[pl.ds(s * RPT, RPT)])
    plsc.subcore_barrier()

    def body(j, carry):
      pltpu.async_copy(table_hbm.at[srcv.at[j]], gbuf, gsem).wait()
      pltpu.sync_copy(gbuf, acc.at[dstv.at[j]], add=True)
      return carry

    lax.fori_loop(0, NCH, body, 0)
    plsc.subcore_barrier()
    pltpu.sync_copy(acc.at[pl.ds(s * RPT, RPT)],
                    out_hbm.at[c, pl.ds(s * RPT, RPT)])

  return sc_scatter


def _sc_degree():
  mesh = plsc.VectorSubcoreMesh(core_axis_name="c", subcore_axis_name="s")

  @functools.partial(
      pl.kernel,
      out_type=jax.ShapeDtypeStruct((2, NP, 16), jnp.float32),
      mesh=mesh,
      scratch_types=[
          pltpu.VMEM((NCH, CHUNK), jnp.int32),
          pltpu.VMEM((CHUNK, 16), jnp.float32),
          pltpu.VMEM_SHARED((NP, 16), jnp.float32),
      ],
  )
  def sc_degree(dst_hbm, ones_hbm, zeros_hbm, out_hbm, dstv, ones_v, acc):
    c = lax.axis_index("c")
    s = lax.axis_index("s")
    w = c * 16 + s
    pltpu.sync_copy(dst_hbm.at[w], dstv)
    pltpu.sync_copy(ones_hbm, ones_v)
    pltpu.sync_copy(zeros_hbm, acc.at[pl.ds(s * RPT, RPT)])
    plsc.subcore_barrier()

    def body(j, carry):
      pltpu.sync_copy(ones_v, acc.at[dstv.at[j]], add=True)
      return carry

    lax.fori_loop(0, NCH, body, 0)
    plsc.subcore_barrier()
    pltpu.sync_copy(acc.at[pl.ds(s * RPT, RPT)],
                    out_hbm.at[c, pl.ds(s * RPT, RPT)])

  return sc_degree


def _dinv_block(deg2_ref, i):
  deg = deg2_ref[0, :, 0] + deg2_ref[1, :, 0] + 1.0
  rows = i * BLK + lax.broadcasted_iota(jnp.int32, (BLK,), 0)
  return jnp.where(rows < N, lax.rsqrt(deg), 0.0)


def _k0_body(x_ref, deg2_ref, w_ref, g_ref):
  dinv = _dinv_block(deg2_ref, pl.program_id(0))
  h = jnp.dot(x_ref[...], w_ref[...], preferred_element_type=jnp.float32)
  g_ref[...] = h * dinv[:, None]


def _kl_body(s_ref, g_ref, deg2_ref, w_ref, b_ref, out_ref):
  dinv = _dinv_block(deg2_ref, pl.program_id(0))
  tot = (s_ref[0] + s_ref[1] + g_ref[...]) * dinv[:, None] + b_ref[...]
  h = jnp.where(tot > 0, tot, 0.01 * tot)
  hw = jnp.dot(h, w_ref[...], preferred_element_type=jnp.float32)
  out_ref[...] = hw * dinv[:, None]


def _k4_body(s_ref, g_ref, deg2_ref, b_ref, wl_ref, bl_ref, conc_ref, log_ref):
  dinv = _dinv_block(deg2_ref, pl.program_id(0))
  tot = (s_ref[0] + s_ref[1] + g_ref[...]) * dinv[:, None] + b_ref[...]
  h = jnp.where(tot > 0, tot, 0.01 * tot)
  conc = jnp.exp(h - jnp.max(h, axis=-1, keepdims=True))
  conc_ref[...] = conc
  log_ref[...] = (
      jnp.dot(conc, wl_ref[...], preferred_element_type=jnp.float32)
      + bl_ref[...])


def _row_spec(d):
  return pl.BlockSpec((BLK, d), lambda i: (i, 0))


def _pair_spec(d):
  return pl.BlockSpec((2, BLK, d), lambda i: (0, i, 0))


def _full_spec(shape):
  nd = len(shape)
  return pl.BlockSpec(shape, lambda i: (0,) * nd)


_GRID = NP // BLK


def _tc_call(body, in_specs, out_specs, out_shape):
  return pl.pallas_call(
      body, grid=(_GRID,), in_specs=in_specs, out_specs=out_specs,
      out_shape=out_shape)


def kernel(x, edge_index, W0, b0, W1, b1, W2, b2, W3, b3, Wl, bl):
  f32 = jnp.float32
  src = edge_index[0]
  dst = edge_index[1]
  pad = EP - E
  src_p = jnp.concatenate(
      [src, jnp.full((pad,), N, jnp.int32)]).reshape(NTILES, NCH, CHUNK)
  dst_p = jnp.concatenate(
      [dst, jnp.zeros((pad,), jnp.int32)]).reshape(NTILES, NCH, CHUNK)
  x_p = jnp.zeros((NP, DIN), f32).at[:N].set(x)
  zeros128 = jnp.zeros((RPT, DH), f32)
  zeros64 = jnp.zeros((RPT, DC), f32)
  zeros16 = jnp.zeros((RPT, 16), f32)
  ones16 = jnp.ones((CHUNK, 16), f32)
  b0r, b1r, b2r, b3r = (b.reshape(1, -1) for b in (b0, b1, b2, b3))
  wl_p = jnp.zeros((DC, 128), f32).at[:, :4].set(Wl)
  bl_p = jnp.zeros((1, 128), f32).at[0, :4].set(bl)

  deg2 = _sc_degree()(dst_p, ones16, zeros16)

  k0 = _tc_call(
      _k0_body,
      [_row_spec(DIN), _pair_spec(16), _full_spec((DIN, DH))],
      _row_spec(DH), jax.ShapeDtypeStruct((NP, DH), f32))
  g0 = k0(x_p, deg2, W0)

  scat128 = _sc_scatter(DH)
  scat64 = _sc_scatter(DC)

  def kl(d_out):
    return _tc_call(
        _kl_body,
        [_pair_spec(DH), _row_spec(DH), _pair_spec(16),
         _full_spec((DH, d_out)), _full_spec((1, DH))],
        _row_spec(d_out), jax.ShapeDtypeStruct((NP, d_out), f32))

  S0 = scat128(src_p, dst_p, g0, zeros128)
  g1 = kl(DH)(S0, g0, deg2, W1, b0r)
  S1 = scat128(src_p, dst_p, g1, zeros128)
  g2 = kl(DH)(S1, g1, deg2, W2, b1r)
  S2 = scat128(src_p, dst_p, g2, zeros128)
  g3 = kl(DC)(S2, g2, deg2, W3, b2r)
  S3 = scat64(src_p, dst_p, g3, zeros64)

  k4 = _tc_call(
      _k4_body,
      [_pair_spec(DC), _row_spec(DC), _pair_spec(16),
       _full_spec((1, DC)), _full_spec((DC, 128)), _full_spec((1, 128))],
      [_row_spec(DC), _row_spec(128)],
      [jax.ShapeDtypeStruct((NP, DC), f32),
       jax.ShapeDtypeStruct((NP, 128), f32)])
  concepts_p, logits_p = k4(S3, g3, deg2, b3r, wl_p, bl_p)
  return concepts_p[:N], logits_p[:N, :4]


# speedup vs baseline: 6.5454x; 6.5454x over previous
import functools

import jax
import jax.numpy as jnp
from jax import lax
from jax.experimental import pallas as pl
from jax.experimental.pallas import tpu as pltpu
from jax.experimental.pallas import tpu_sc as plsc

N = 10000
E = 320000
DIN = 128
DH = 128
DC = 64

NP = 10240
BLK = 512
NTILES = 32
CHUNK = 128
NCH = 80
EPT = NCH * CHUNK
EP = NTILES * EPT
RPT = NP // 16


def _sc_scatter(d):
  mesh = plsc.VectorSubcoreMesh(core_axis_name="c", subcore_axis_name="s")

  @functools.partial(
      pl.kernel,
      out_type=jax.ShapeDtypeStruct((2, NP, d), jnp.float32),
      mesh=mesh,
      scratch_types=[
          pltpu.VMEM((NCH, CHUNK), jnp.int32),
          pltpu.VMEM((NCH, CHUNK), jnp.int32),
          pltpu.VMEM((CHUNK, d), jnp.float32),
          pltpu.SemaphoreType.DMA,
          pltpu.VMEM_SHARED((NP, d), jnp.float32),
      ],
  )
  def sc_scatter(src_hbm, dst_hbm, table_hbm, zeros_hbm, out_hbm,
                 srcv, dstv, gbuf, gsem, acc):
    c = lax.axis_index("c")
    s = lax.axis_index("s")
    w = c * 16 + s
    pltpu.sync_copy(src_hbm.at[w], srcv)
    pltpu.sync_copy(dst_hbm.at[w], dstv)
    pltpu.sync_copy(zeros_hbm, acc.at[pl.ds(s * RPT, RPT)])
    plsc.subcore_barrier()

    def body(j, carry):
      pltpu.async_copy(table_hbm.at[srcv.at[j]], gbuf, gsem).wait()
      pltpu.sync_copy(gbuf, acc.at[dstv.at[j]], add=True)
      return carry

    lax.fori_loop(0, NCH, body, 0)
    plsc.subcore_barrier()
    pltpu.sync_copy(acc.at[pl.ds(s * RPT, RPT)],
                    out_hbm.at[c, pl.ds(s * RPT, RPT)])

  return sc_scatter


def _sc_degree():
  mesh = plsc.VectorSubcoreMesh(core_axis_name="c", subcore_axis_name="s")

  @functools.partial(
      pl.kernel,
      out_type=jax.ShapeDtypeStruct((2, NP, 16), jnp.float32),
      mesh=mesh,
      scratch_types=[
          pltpu.VMEM((NCH, CHUNK), jnp.int32),
          pltpu.VMEM((CHUNK, 16), jnp.float32),
          pltpu.VMEM_SHARED((NP, 16), jnp.float32),
      ],
  )
  def sc_degree(dst_hbm, ones_hbm, zeros_hbm, out_hbm, dstv, ones_v, acc):
    c = lax.axis_index("c")
    s = lax.axis_index("s")
    w = c * 16 + s
    pltpu.sync_copy(dst_hbm.at[w], dstv)
    pltpu.sync_copy(ones_hbm, ones_v)
    pltpu.sync_copy(zeros_hbm, acc.at[pl.ds(s * RPT, RPT)])
    plsc.subcore_barrier()

    def body(j, carry):
      pltpu.sync_copy(ones_v, acc.at[dstv.at[j]], add=True)
      return carry

    lax.fori_loop(0, NCH, body, 0)
    plsc.subcore_barrier()
    pltpu.sync_copy(acc.at[pl.ds(s * RPT, RPT)],
                    out_hbm.at[c, pl.ds(s * RPT, RPT)])

  return sc_degree


def _dinv_block(deg2_ref, i):
  deg = deg2_ref[0, :, 0] + deg2_ref[1, :, 0] + 1.0
  rows = i * BLK + lax.broadcasted_iota(jnp.int32, (BLK,), 0)
  return jnp.where(rows < N, lax.rsqrt(deg), 0.0)


def _k0_body(x_ref, deg2_ref, w_ref, g_ref):
  dinv = _dinv_block(deg2_ref, pl.program_id(0))
  h = jnp.dot(x_ref[...], w_ref[...], preferred_element_type=jnp.float32)
  g_ref[...] = h * dinv[:, None]


def _kl_body(s_ref, g_ref, deg2_ref, w_ref, b_ref, out_ref):
  dinv = _dinv_block(deg2_ref, pl.program_id(0))
  tot = (s_ref[0] + s_ref[1] + g_ref[...]) * dinv[:, None] + b_ref[...]
  h = jnp.where(tot > 0, tot, 0.01 * tot)
  hw = jnp.dot(h, w_ref[...], preferred_element_type=jnp.float32)
  out_ref[...] = hw * dinv[:, None]


def _k4_body(s_ref, g_ref, deg2_ref, b_ref, wl_ref, bl_ref, conc_ref, log_ref):
  dinv = _dinv_block(deg2_ref, pl.program_id(0))
  tot = (s_ref[0, :, :DC] + s_ref[1, :, :DC] + g_ref[:, :DC]) * dinv[:, None]
  tot = tot + b_ref[:, :DC]
  h = jnp.where(tot > 0, tot, 0.01 * tot)
  conc = jnp.exp(h - jnp.max(h, axis=-1, keepdims=True))
  conc_ref[...] = conc
  log_ref[...] = (
      jnp.dot(conc, wl_ref[...], preferred_element_type=jnp.float32)
      + bl_ref[...])


def _row_spec(d):
  return pl.BlockSpec((BLK, d), lambda i: (i, 0))


def _pair_spec(d):
  return pl.BlockSpec((2, BLK, d), lambda i: (0, i, 0))


def _full_spec(shape):
  nd = len(shape)
  return pl.BlockSpec(shape, lambda i: (0,) * nd)


_GRID = NP // BLK


def _tc_call(body, in_specs, out_specs, out_shape):
  return pl.pallas_call(
      body, grid=(_GRID,), in_specs=in_specs, out_specs=out_specs,
      out_shape=out_shape)


def kernel(x, edge_index, W0, b0, W1, b1, W2, b2, W3, b3, Wl, bl):
  f32 = jnp.float32
  src = edge_index[0]
  dst = edge_index[1]
  pad = EP - E
  src_p = jnp.concatenate(
      [src, jnp.full((pad,), N, jnp.int32)]).reshape(NTILES, NCH, CHUNK)
  dst_p = jnp.concatenate(
      [dst, jnp.zeros((pad,), jnp.int32)]).reshape(NTILES, NCH, CHUNK)
  x_p = jnp.zeros((NP, DIN), f32).at[:N].set(x)
  zeros128 = jnp.zeros((RPT, DH), f32)
  zeros16 = jnp.zeros((RPT, 16), f32)
  ones16 = jnp.ones((CHUNK, 16), f32)
  b0r, b1r, b2r = (b.reshape(1, -1) for b in (b0, b1, b2))
  b3r = jnp.zeros((1, DH), f32).at[0, :DC].set(b3)
  w3_p = jnp.zeros((DH, DH), f32).at[:, :DC].set(W3)
  wl_p = jnp.zeros((DC, 128), f32).at[:, :4].set(Wl)
  bl_p = jnp.zeros((1, 128), f32).at[0, :4].set(bl)

  deg2 = _sc_degree()(dst_p, ones16, zeros16)

  k0 = _tc_call(
      _k0_body,
      [_row_spec(DIN), _pair_spec(16), _full_spec((DIN, DH))],
      _row_spec(DH), jax.ShapeDtypeStruct((NP, DH), f32))
  g0 = k0(x_p, deg2, W0)

  scat128 = _sc_scatter(DH)

  def kl(d_out):
    return _tc_call(
        _kl_body,
        [_pair_spec(DH), _row_spec(DH), _pair_spec(16),
         _full_spec((DH, d_out)), _full_spec((1, DH))],
        _row_spec(d_out), jax.ShapeDtypeStruct((NP, d_out), f32))

  S0 = scat128(src_p, dst_p, g0, zeros128)
  g1 = kl(DH)(S0, g0, deg2, W1, b0r)
  S1 = scat128(src_p, dst_p, g1, zeros128)
  g2 = kl(DH)(S1, g1, deg2, W2, b1r)
  S2 = scat128(src_p, dst_p, g2, zeros128)
  g3 = kl(DH)(S2, g2, deg2, w3_p, b2r)
  S3 = scat128(src_p, dst_p, g3, zeros128)

  k4 = _tc_call(
      _k4_body,
      [_pair_spec(DH), _row_spec(DH), _pair_spec(16),
       _full_spec((1, DH)), _full_spec((DC, 128)), _full_spec((1, 128))],
      [_row_spec(DC), _row_spec(128)],
      [jax.ShapeDtypeStruct((NP, DC), f32),
       jax.ShapeDtypeStruct((NP, 128), f32)])
  concepts_p, logits_p = k4(S3, g3, deg2, b3r, wl_p, bl_p)
  return concepts_p[:N], logits_p[:N, :4]


# speedup vs baseline: 6.5522x; 1.0010x over previous
import functools

import jax
import jax.numpy as jnp
from jax import lax
from jax.experimental import pallas as pl
from jax.experimental.pallas import tpu as pltpu
from jax.experimental.pallas import tpu_sc as plsc

N = 10000
E = 320000
DIN = 128
DH = 128
DC = 64

NP = 10240
BLK = 512
NTILES = 32
CHUNK = 128
NCH = 80
EPT = NCH * CHUNK
EP = NTILES * EPT
RPT = NP // 16


def _sc_scatter(d):
  mesh = plsc.VectorSubcoreMesh(core_axis_name="c", subcore_axis_name="s")

  @functools.partial(
      pl.kernel,
      out_type=jax.ShapeDtypeStruct((2, NP, d), jnp.float32),
      mesh=mesh,
      scratch_types=[
          pltpu.VMEM((NCH, CHUNK), jnp.int32),
          pltpu.VMEM((NCH, CHUNK), jnp.int32),
          pltpu.VMEM((CHUNK, d), jnp.float32),
          pltpu.SemaphoreType.DMA,
          pltpu.VMEM_SHARED((NP, d), jnp.float32),
      ],
  )
  def sc_scatter(src_hbm, dst_hbm, table_hbm, zeros_hbm, out_hbm,
                 srcv, dstv, gbuf, sem0, acc):
    c = lax.axis_index("c")
    s = lax.axis_index("s")
    w = c * 16 + s
    pltpu.sync_copy(src_hbm.at[w], srcv)
    pltpu.sync_copy(dst_hbm.at[w], dstv)
    pltpu.sync_copy(zeros_hbm, acc.at[pl.ds(s * RPT, RPT)])
    plsc.subcore_barrier()

    def body(j, carry):
      pltpu.async_copy(table_hbm.at[srcv.at[j]], gbuf, sem0).wait()
      pltpu.sync_copy(gbuf, acc.at[dstv.at[j]], add=True)
      return carry

    lax.fori_loop(0, NCH, body, 0)
    plsc.subcore_barrier()
    pltpu.sync_copy(acc.at[pl.ds(s * RPT, RPT)],
                    out_hbm.at[c, pl.ds(s * RPT, RPT)])

  return sc_scatter


def _sc_degree():
  mesh = plsc.VectorSubcoreMesh(core_axis_name="c", subcore_axis_name="s")

  @functools.partial(
      pl.kernel,
      out_type=jax.ShapeDtypeStruct((2, NP, 16), jnp.float32),
      mesh=mesh,
      scratch_types=[
          pltpu.VMEM((NCH, CHUNK), jnp.int32),
          pltpu.VMEM((CHUNK, 16), jnp.float32),
          pltpu.VMEM_SHARED((NP, 16), jnp.float32),
      ],
  )
  def sc_degree(dst_hbm, ones_hbm, zeros_hbm, out_hbm, dstv, ones_v, acc):
    c = lax.axis_index("c")
    s = lax.axis_index("s")
    w = c * 16 + s
    pltpu.sync_copy(dst_hbm.at[w], dstv)
    pltpu.sync_copy(ones_hbm, ones_v)
    pltpu.sync_copy(zeros_hbm, acc.at[pl.ds(s * RPT, RPT)])
    plsc.subcore_barrier()

    def body(j, carry):
      pltpu.sync_copy(ones_v, acc.at[dstv.at[j]], add=True)
      return carry

    lax.fori_loop(0, NCH, body, 0)
    plsc.subcore_barrier()
    pltpu.sync_copy(acc.at[pl.ds(s * RPT, RPT)],
                    out_hbm.at[c, pl.ds(s * RPT, RPT)])

  return sc_degree


def _dinv_block(deg2_ref, i):
  deg = deg2_ref[0, :, 0] + deg2_ref[1, :, 0] + 1.0
  rows = i * BLK + lax.broadcasted_iota(jnp.int32, (BLK,), 0)
  return jnp.where(rows < N, lax.rsqrt(deg), 0.0)


def _k0_body(x_ref, deg2_ref, w_ref, g_ref):
  dinv = _dinv_block(deg2_ref, pl.program_id(0))
  h = jnp.dot(x_ref[...], w_ref[...], preferred_element_type=jnp.float32)
  g_ref[...] = h * dinv[:, None]


def _kl_body(s_ref, g_ref, deg2_ref, w_ref, b_ref, out_ref):
  dinv = _dinv_block(deg2_ref, pl.program_id(0))
  tot = (s_ref[0] + s_ref[1] + g_ref[...]) * dinv[:, None] + b_ref[...]
  h = jnp.where(tot > 0, tot, 0.01 * tot)
  hw = jnp.dot(h, w_ref[...], preferred_element_type=jnp.float32)
  out_ref[...] = hw * dinv[:, None]


def _k4_body(s_ref, g_ref, deg2_ref, b_ref, wl_ref, bl_ref, conc_ref, log_ref):
  dinv = _dinv_block(deg2_ref, pl.program_id(0))
  tot = (s_ref[0, :, :DC] + s_ref[1, :, :DC] + g_ref[:, :DC]) * dinv[:, None]
  tot = tot + b_ref[:, :DC]
  h = jnp.where(tot > 0, tot, 0.01 * tot)
  conc = jnp.exp(h - jnp.max(h, axis=-1, keepdims=True))
  conc_ref[...] = conc
  log_ref[...] = (
      jnp.dot(conc, wl_ref[...], preferred_element_type=jnp.float32)
      + bl_ref[...])


def _row_spec(d):
  return pl.BlockSpec((BLK, d), lambda i: (i, 0))


def _pair_spec(d):
  return pl.BlockSpec((2, BLK, d), lambda i: (0, i, 0))


def _full_spec(shape):
  nd = len(shape)
  return pl.BlockSpec(shape, lambda i: (0,) * nd)


_GRID = NP // BLK


def _tc_call(body, in_specs, out_specs, out_shape):
  return pl.pallas_call(
      body, grid=(_GRID,), in_specs=in_specs, out_specs=out_specs,
      out_shape=out_shape)


def kernel(x, edge_index, W0, b0, W1, b1, W2, b2, W3, b3, Wl, bl):
  f32 = jnp.float32
  src = edge_index[0]
  dst = edge_index[1]
  pad = EP - E
  src_p = jnp.concatenate(
      [src, jnp.full((pad,), N, jnp.int32)]).reshape(NTILES, NCH, CHUNK)
  pad_dst = jnp.tile(jnp.arange(CHUNK, dtype=jnp.int32), pad // CHUNK)
  dst_p = jnp.concatenate([dst, pad_dst]).reshape(NTILES, NCH, CHUNK)
  x_p = jnp.zeros((NP, DIN), f32).at[:N].set(x)
  zeros128 = jnp.zeros((RPT, DH), f32)
  zeros16 = jnp.zeros((RPT, 16), f32)
  ones16 = jnp.ones((CHUNK, 16), f32)
  b0r, b1r, b2r = (b.reshape(1, -1) for b in (b0, b1, b2))
  b3r = jnp.zeros((1, DH), f32).at[0, :DC].set(b3)
  w3_p = jnp.zeros((DH, DH), f32).at[:, :DC].set(W3)
  wl_p = jnp.zeros((DC, 128), f32).at[:, :4].set(Wl)
  bl_p = jnp.zeros((1, 128), f32).at[0, :4].set(bl)

  deg2 = _sc_degree()(dst_p, ones16, zeros16)

  k0 = _tc_call(
      _k0_body,
      [_row_spec(DIN), _pair_spec(16), _full_spec((DIN, DH))],
      _row_spec(DH), jax.ShapeDtypeStruct((NP, DH), f32))
  g0 = k0(x_p, deg2, W0)

  scat128 = _sc_scatter(DH)

  def kl(d_out):
    return _tc_call(
        _kl_body,
        [_pair_spec(DH), _row_spec(DH), _pair_spec(16),
         _full_spec((DH, d_out)), _full_spec((1, DH))],
        _row_spec(d_out), jax.ShapeDtypeStruct((NP, d_out), f32))

  S0 = scat128(src_p, dst_p, g0, zeros128)
  g1 = kl(DH)(S0, g0, deg2, W1, b0r)
  S1 = scat128(src_p, dst_p, g1, zeros128)
  g2 = kl(DH)(S1, g1, deg2, W2, b1r)
  S2 = scat128(src_p, dst_p, g2, zeros128)
  g3 = kl(DH)(S2, g2, deg2, w3_p, b2r)
  S3 = scat128(src_p, dst_p, g3, zeros128)

  k4 = _tc_call(
      _k4_body,
      [_pair_spec(DH), _row_spec(DH), _pair_spec(16),
       _full_spec((1, DH)), _full_spec((DC, 128)), _full_spec((1, 128))],
      [_row_spec(DC), _row_spec(128)],
      [jax.ShapeDtypeStruct((NP, DC), f32),
       jax.ShapeDtypeStruct((NP, 128), f32)])
  concepts_p, logits_p = k4(S3, g3, deg2, b3r, wl_p, bl_p)
  return concepts_p[:N], logits_p[:N, :4]
